# Initial kernel scaffold; baseline (speedup 1.0000x reference)
import jax
import jax.numpy as jnp
from jax.experimental import pallas as pl


def kernel(image_real, image_imag, coord):
    raise NotImplementedError("write your pallas kernel here")


# speedup vs baseline: 33.4169x; 33.4169x over previous
import functools
import math

import numpy as np

import jax
import jax.numpy as jnp
from jax import lax
from jax.experimental import pallas as pl
from jax.experimental.pallas import tpu as pltpu
from jax.experimental.pallas import tpu_sc as plsc

SHAPE = (256, 256)
OVERSAMP = 1.125
WIDTH = 3
OS = tuple(int(np.ceil(OVERSAMP * n)) for n in SHAPE)
BETA = float(np.pi * (((WIDTH / OVERSAMP) * (OVERSAMP - 0.5)) ** 2 - 0.8) ** 0.5)
I0BETA = float(np.i0(BETA))
NCOIL = 8
NSAMP = 262144

NC, NS = 2, 16
NTILE = NC * NS
SPT = NSAMP // NTILE
CH = 128
NCHUNK = SPT // CH
NROW = NSAMP // CH
GRIDPTS = OS[0] * OS[1]


def _make_table(image_real, image_imag):
    img = image_real + 1j * image_imag
    for d in range(2):
        n = SHAPE[d]
        idx = jnp.arange(n, dtype=jnp.float32)
        a = jnp.sqrt(BETA ** 2 - (np.pi * WIDTH * (idx - n // 2) / OS[d]) ** 2)
        apod = a / jnp.sinh(a)
        bshape = [1] * img.ndim
        bshape[d - 2] = n
        img = img * apod.reshape(bshape)
    pads = [(0, 0)]
    for d in range(2):
        total = OS[d] - SHAPE[d]
        pads.append((total // 2, total - total // 2))
    img = jnp.pad(img, pads)
    grid = jnp.fft.fftshift(
        jnp.fft.fft2(jnp.fft.ifftshift(img, axes=(-2, -1)), axes=(-2, -1),
                     norm='ortho'),
        axes=(-2, -1))
    gf = grid.reshape(NCOIL, GRIDPTS)
    return jnp.concatenate([jnp.real(gf), jnp.imag(gf)], axis=0).T


_KB_COEF = [float((BETA * BETA / 4.0) ** k
                  / (math.factorial(k) ** 2) / np.i0(BETA))
            for k in range(11)]


def _kb(x):
    u = jnp.clip(1.0 - (2.0 * x / WIDTH) ** 2, 0.0, None)
    acc = jnp.full_like(u, _KB_COEF[10])
    for k in range(9, -1, -1):
        acc = acc * u + _KB_COEF[k]
    return acc


def _tc_body(cx_ref, cy_ref, idx_ref, w_ref):
    sx = float(OS[0]) / float(SHAPE[0])
    sy = float(OS[1]) / float(SHAPE[1])
    cx = cx_ref[...] * sx + float(OS[0] // 2)
    cy = cy_ref[...] * sy + float(OS[1] // 2)
    bx = jnp.floor(cx)
    by = jnp.floor(cy)
    fx = cx - bx
    fy = cy - by
    bxi = bx.astype(jnp.int32)
    byi = by.astype(jnp.int32)
    wxs, wys, ixs, iys = [], [], [], []
    for o in range(WIDTH):
        d = float(o - WIDTH // 2)
        wxs.append(_kb(fx - d))
        wys.append(_kb(fy - d))
        ixs.append(jnp.mod(bxi + (o - WIDTH // 2), OS[0]))
        iys.append(jnp.mod(byi + (o - WIDTH // 2), OS[1]))
    for a in range(WIDTH):
        for b in range(WIDTH):
            j = a * WIDTH + b
            idx_ref[:, j, :] = ixs[a] * OS[1] + iys[b]
            w_ref[:, j, :] = wxs[a] * wys[b]


def _tc_weights(coord):
    cx = coord[:, 0].reshape(NROW, 128)
    cy = coord[:, 1].reshape(NROW, 128)
    rb = 256
    grid = (NROW // rb,)
    return pl.pallas_call(
        _tc_body,
        grid=grid,
        in_specs=[
            pl.BlockSpec((rb, 128), lambda i: (i, 0)),
            pl.BlockSpec((rb, 128), lambda i: (i, 0)),
        ],
        out_specs=[
            pl.BlockSpec((rb, 9, 128), lambda i: (i, 0, 0)),
            pl.BlockSpec((rb, 9, 128), lambda i: (i, 0, 0)),
        ],
        out_shape=[
            jax.ShapeDtypeStruct((NROW, 9, 128), jnp.int32),
            jax.ShapeDtypeStruct((NROW, 9, 128), jnp.float32),
        ],
    )(cx, cy)


def _sc_body(idx_hbm, w_hbm, table_hbm, out_hbm, idx_v, w_v, rows_v, out_v,
             sem_g):
    wid = lax.axis_index("s") * NC + lax.axis_index("c")
    row0 = wid * NCHUNK

    def chunk_body(k, _):
        r = row0 + k
        pltpu.sync_copy(idx_hbm.at[r], idx_v.at[0])
        pltpu.sync_copy(w_hbm.at[r], w_v.at[0])
        cps = [
            pltpu.async_copy(table_hbm.at[idx_v.at[0, j]], rows_v.at[0, j],
                             sem_g)
            for j in range(9)
        ]
        for cp in cps:
            cp.wait()

        def g_body(g, _):
            l0 = g * 16
            lanes = l0 + lax.iota(jnp.int32, 16)
            wv = [w_v[0, j, pl.ds(l0, 16)] for j in range(9)]

            zero16 = jnp.zeros((16,), dtype=jnp.int32)

            def c_body(c, _):
                cvec = jnp.full((16,), c, dtype=jnp.int32)

                def tap(j):
                    jvec = jnp.full((16,), j, dtype=jnp.int32)
                    return plsc.load_gather(rows_v, [zero16, jvec, lanes, cvec])

                acc = wv[0] * tap(0)
                for j in range(1, 9):
                    acc = acc + wv[j] * tap(j)
                out_v[0, c, pl.ds(l0, 16)] = acc
                return 0

            lax.fori_loop(0, 16, c_body, 0, unroll=True)
            return 0

        lax.fori_loop(0, CH // 16, g_body, 0)
        s0 = wid * SPT + k * CH
        pltpu.sync_copy(out_v.at[0], out_hbm.at[:, pl.ds(s0, CH)])
        return 0

    lax.fori_loop(0, NCHUNK, chunk_body, 0)


@functools.lru_cache(maxsize=1)
def _get_sc_interp():
    return pl.kernel(
        _sc_body,
        out_type=jax.ShapeDtypeStruct((16, NSAMP), jnp.float32),
        mesh=plsc.VectorSubcoreMesh(core_axis_name="c", subcore_axis_name="s",
                                    num_cores=NC, num_subcores=NS),
        compiler_params=pltpu.CompilerParams(needs_layout_passes=False,
                                             use_tc_tiling_on_sc=False),
        scratch_types=[
            pltpu.VMEM((1, 9, CH), jnp.int32),
            pltpu.VMEM((1, 9, CH), jnp.float32),
            pltpu.VMEM((1, 9, CH, 16), jnp.float32),
            pltpu.VMEM((1, 16, CH), jnp.float32),
            pltpu.SemaphoreType.DMA,
        ],
    )


@jax.jit
def kernel(image_real, image_imag, coord):
    table = _make_table(image_real, image_imag)
    idx, wgt = _tc_weights(coord)
    out = _get_sc_interp()(idx, wgt, table)
    return out.reshape(2, NCOIL, NSAMP)


# speedup vs baseline: 42.8141x; 1.2812x over previous
import functools
import math

import numpy as np

import jax
import jax.numpy as jnp
from jax import lax
from jax.experimental import pallas as pl
from jax.experimental.pallas import tpu as pltpu
from jax.experimental.pallas import tpu_sc as plsc

SHAPE = (256, 256)
OVERSAMP = 1.125
WIDTH = 3
OS = tuple(int(np.ceil(OVERSAMP * n)) for n in SHAPE)
BETA = float(np.pi * (((WIDTH / OVERSAMP) * (OVERSAMP - 0.5)) ** 2 - 0.8) ** 0.5)
I0BETA = float(np.i0(BETA))
NCOIL = 8
NSAMP = 262144

NC, NS = 2, 16
NTILE = NC * NS
SPT = NSAMP // NTILE
CH = 128
NCHUNK = SPT // CH
NROW = NSAMP // CH
GRIDPTS = OS[0] * OS[1]


def _make_table(image_real, image_imag):
    img = image_real + 1j * image_imag
    for d in range(2):
        n = SHAPE[d]
        idx = jnp.arange(n, dtype=jnp.float32)
        a = jnp.sqrt(BETA ** 2 - (np.pi * WIDTH * (idx - n // 2) / OS[d]) ** 2)
        apod = a / jnp.sinh(a)
        bshape = [1] * img.ndim
        bshape[d - 2] = n
        img = img * apod.reshape(bshape)
    pads = [(0, 0)]
    for d in range(2):
        total = OS[d] - SHAPE[d]
        pads.append((total // 2, total - total // 2))
    img = jnp.pad(img, pads)
    grid = jnp.fft.fftshift(
        jnp.fft.fft2(jnp.fft.ifftshift(img, axes=(-2, -1)), axes=(-2, -1),
                     norm='ortho'),
        axes=(-2, -1))
    gf = grid.reshape(NCOIL, GRIDPTS)
    return jnp.concatenate([jnp.real(gf), jnp.imag(gf)], axis=0).T


_KB_COEF = [float((BETA * BETA / 4.0) ** k
                  / (math.factorial(k) ** 2) / np.i0(BETA))
            for k in range(11)]


def _kb(x):
    u = jnp.clip(1.0 - (2.0 * x / WIDTH) ** 2, 0.0, None)
    acc = jnp.full_like(u, _KB_COEF[10])
    for k in range(9, -1, -1):
        acc = acc * u + _KB_COEF[k]
    return acc


def _tc_body(cx_ref, cy_ref, idx_ref, w_ref):
    sx = float(OS[0]) / float(SHAPE[0])
    sy = float(OS[1]) / float(SHAPE[1])
    cx = cx_ref[...] * sx + float(OS[0] // 2)
    cy = cy_ref[...] * sy + float(OS[1] // 2)
    bx = jnp.floor(cx)
    by = jnp.floor(cy)
    fx = cx - bx
    fy = cy - by
    bxi = bx.astype(jnp.int32)
    byi = by.astype(jnp.int32)
    wxs, wys, ixs, iys = [], [], [], []
    for o in range(WIDTH):
        d = float(o - WIDTH // 2)
        wxs.append(_kb(fx - d))
        wys.append(_kb(fy - d))
        ixs.append(jnp.mod(bxi + (o - WIDTH // 2), OS[0]))
        iys.append(jnp.mod(byi + (o - WIDTH // 2), OS[1]))
    for a in range(WIDTH):
        for b in range(WIDTH):
            j = a * WIDTH + b
            idx_ref[:, j, :] = ixs[a] * OS[1] + iys[b]
            w_ref[:, j, :] = wxs[a] * wys[b]


def _tc_weights(coord):
    cx = coord[:, 0].reshape(NROW, 128)
    cy = coord[:, 1].reshape(NROW, 128)
    rb = 256
    grid = (NROW // rb,)
    return pl.pallas_call(
        _tc_body,
        grid=grid,
        in_specs=[
            pl.BlockSpec((rb, 128), lambda i: (i, 0)),
            pl.BlockSpec((rb, 128), lambda i: (i, 0)),
        ],
        out_specs=[
            pl.BlockSpec((rb, 9, 128), lambda i: (i, 0, 0)),
            pl.BlockSpec((rb, 9, 128), lambda i: (i, 0, 0)),
        ],
        out_shape=[
            jax.ShapeDtypeStruct((NROW, 9, 128), jnp.int32),
            jax.ShapeDtypeStruct((NROW, 9, 128), jnp.float32),
        ],
    )(cx, cy)


def _sc_body(idx_hbm, w_hbm, table_hbm, out_hbm, idx_v, w_v, rows_v, out_v,
             sem_idx, sem_w, sem_g, sem_out):
    wid = lax.axis_index("s") * NC + lax.axis_index("c")
    row0 = wid * NCHUNK
    s_base = wid * SPT

    def fire_gathers(b):
        for j in range(9):
            pltpu.async_copy(table_hbm.at[idx_v.at[b, j]], rows_v.at[b, j],
                             sem_g)

    def drain_gathers(b):
        for j in range(9):
            pltpu.make_async_copy(table_hbm.at[idx_v.at[b, j]],
                                  rows_v.at[b, j], sem_g).wait()

    def compute(b):
        def g_body(g, _):
            l0 = g * 16
            lanes = l0 + lax.iota(jnp.int32, 16)
            wv = [w_v[b, j, pl.ds(l0, 16)] for j in range(9)]
            bvec = jnp.full((16,), b, dtype=jnp.int32)

            def c_body(c, _):
                cvec = jnp.full((16,), c, dtype=jnp.int32)

                def tap(j):
                    jvec = jnp.full((16,), j, dtype=jnp.int32)
                    return plsc.load_gather(rows_v, [bvec, jvec, lanes, cvec])

                acc = wv[0] * tap(0)
                for j in range(1, 9):
                    acc = acc + wv[j] * tap(j)
                out_v[b, c, pl.ds(l0, 16)] = acc
                return 0

            lax.fori_loop(0, 16, c_body, 0, unroll=True)
            return 0

        lax.fori_loop(0, CH // 16, g_body, 0)

    pltpu.sync_copy(idx_hbm.at[row0], idx_v.at[0])
    pltpu.sync_copy(w_hbm.at[row0], w_v.at[0])
    pltpu.sync_copy(idx_hbm.at[row0 + 1], idx_v.at[1])
    pltpu.sync_copy(w_hbm.at[row0 + 1], w_v.at[1])
    fire_gathers(0)

    def pair_body(kp, _):
        for p in range(2):
            k = 2 * kp + p
            b = p
            o = 1 - p
            drain_gathers(b)

            @pl.when(k + 1 < NCHUNK)
            def _():
                @pl.when(k >= 1)
                def _():
                    pltpu.make_async_copy(idx_hbm.at[row0 + k + 1],
                                          idx_v.at[o], sem_idx).wait()

                fire_gathers(o)

            @pl.when(k + 2 < NCHUNK)
            def _():
                pltpu.async_copy(idx_hbm.at[row0 + k + 2], idx_v.at[b],
                                 sem_idx)

            compute(b)

            @pl.when(k >= 1)
            def _():
                pltpu.make_async_copy(
                    out_v.at[o],
                    out_hbm.at[:, pl.ds(s_base + (k - 1) * CH, CH)],
                    sem_out).wait()

            pltpu.async_copy(out_v.at[b],
                             out_hbm.at[:, pl.ds(s_base + k * CH, CH)],
                             sem_out)

            @pl.when((k >= 1) & (k + 1 < NCHUNK))
            def _():
                pltpu.make_async_copy(w_hbm.at[row0 + k + 1], w_v.at[o],
                                      sem_w).wait()

            @pl.when(k + 2 < NCHUNK)
            def _():
                pltpu.async_copy(w_hbm.at[row0 + k + 2], w_v.at[b], sem_w)

        return 0

    lax.fori_loop(0, NCHUNK // 2, pair_body, 0)
    pltpu.make_async_copy(
        out_v.at[1],
        out_hbm.at[:, pl.ds(s_base + (NCHUNK - 1) * CH, CH)],
        sem_out).wait()


@functools.lru_cache(maxsize=1)
def _get_sc_interp():
    return pl.kernel(
        _sc_body,
        out_type=jax.ShapeDtypeStruct((16, NSAMP), jnp.float32),
        mesh=plsc.VectorSubcoreMesh(core_axis_name="c", subcore_axis_name="s",
                                    num_cores=NC, num_subcores=NS),
        compiler_params=pltpu.CompilerParams(needs_layout_passes=False,
                                             use_tc_tiling_on_sc=False),
        scratch_types=[
            pltpu.VMEM((2, 9, CH), jnp.int32),
            pltpu.VMEM((2, 9, CH), jnp.float32),
            pltpu.VMEM((2, 9, CH, 16), jnp.float32),
            pltpu.VMEM((2, 16, CH), jnp.float32),
            pltpu.SemaphoreType.DMA,
            pltpu.SemaphoreType.DMA,
            pltpu.SemaphoreType.DMA,
            pltpu.SemaphoreType.DMA,
        ],
    )


@jax.jit
def kernel(image_real, image_imag, coord):
    table = _make_table(image_real, image_imag)
    idx, wgt = _tc_weights(coord)
    out = _get_sc_interp()(idx, wgt, table)
    return out.reshape(2, NCOIL, NSAMP)
